# Initial kernel scaffold; baseline (speedup 1.0000x reference)
import jax
import jax.numpy as jnp
from jax.experimental import pallas as pl


def kernel(x, edge_index, enc_W0, enc_b0, enc_W1, enc_b1, enc_W2, enc_b2, enc_W3, enc_b3, dec_W0, dec_b0, dec_W1, dec_b1, dec_W2, dec_b2, dec_W3, dec_b3):
    raise NotImplementedError("write your pallas kernel here")


# speedup vs baseline: 2.7766x; 2.7766x over previous
import functools

import jax
import jax.numpy as jnp
from jax import lax
from jax.experimental import pallas as pl
from jax.experimental.pallas import tpu as pltpu
from jax.experimental.pallas import tpu_sc as plsc

_NC = 2
_NS = 16
_NW = _NC * _NS
_C = 80
_G = 5

_BN = 2000
_BE = 8000


def _dot(a, b):
    return lax.dot_general(a, b, (((1,), (0,)), ((), ())),
                           precision=lax.Precision.HIGHEST,
                           preferred_element_type=jnp.float32)


def _node_tables_tc(x, wa, wb, b0):
    nn, d = x.shape
    k = wa.shape[1]

    def body(x_ref, wa_ref, wb_ref, b0_ref, u_ref, v_ref):
        xb = x_ref[...]
        u_ref[...] = _dot(xb, wa_ref[...]) + b0_ref[...]
        v_ref[...] = _dot(xb, wb_ref[...])

    return pl.pallas_call(
        body,
        grid=(nn // _BN,),
        in_specs=[
            pl.BlockSpec((_BN, d), lambda i: (i, 0)),
            pl.BlockSpec((d, k), lambda i: (0, 0)),
            pl.BlockSpec((d, k), lambda i: (0, 0)),
            pl.BlockSpec((1, k), lambda i: (0, 0)),
        ],
        out_specs=[
            pl.BlockSpec((_BN, k), lambda i: (i, 0)),
            pl.BlockSpec((_BN, k), lambda i: (i, 0)),
        ],
        out_shape=[
            jax.ShapeDtypeStruct((nn, k), jnp.float32),
            jax.ShapeDtypeStruct((nn, k), jnp.float32),
        ],
    )(x, wa, wb, b0)


def _edge_mlp_tc(gu, gv, w1, b1, w2, b2, w3=None, b3=None):
    ne, k = gu.shape
    k1 = w1.shape[1]
    k2 = w2.shape[1]
    kout = w3.shape[1] if w3 is not None else k2
    three = w3 is not None

    def body(gu_ref, gv_ref, w1_ref, b1_ref, w2_ref, b2_ref, *rest):
        out_ref = rest[-1]
        h = jnp.maximum(gu_ref[...] + gv_ref[...], 0.0)
        h = jnp.maximum(_dot(h, w1_ref[...]) + b1_ref[...], 0.0)
        h = jnp.maximum(_dot(h, w2_ref[...]) + b2_ref[...], 0.0)
        if three:
            h = jnp.maximum(_dot(h, rest[0][...]) + rest[1][...], 0.0)
        out_ref[...] = h

    in_specs = [
        pl.BlockSpec((_BE, k), lambda i: (i, 0)),
        pl.BlockSpec((_BE, k), lambda i: (i, 0)),
        pl.BlockSpec((k, k1), lambda i: (0, 0)),
        pl.BlockSpec((1, k1), lambda i: (0, 0)),
        pl.BlockSpec((k1, k2), lambda i: (0, 0)),
        pl.BlockSpec((1, k2), lambda i: (0, 0)),
    ]
    args = [gu, gv, w1, b1, w2, b2]
    if three:
        in_specs += [pl.BlockSpec((k2, kout), lambda i: (0, 0)),
                     pl.BlockSpec((1, kout), lambda i: (0, 0))]
        args += [w3, b3]
    return pl.pallas_call(
        body,
        grid=(ne // _BE,),
        in_specs=in_specs,
        out_specs=pl.BlockSpec((_BE, kout), lambda i: (i, 0)),
        out_shape=jax.ShapeDtypeStruct((ne, kout), jnp.float32),
    )(*args)


def _dec_tables_tc(p, ea, eb, db0):
    nn = p.shape[1]
    lat = ea.shape[0]
    k = ea.shape[1]
    d = p.shape[2]

    def body(p_ref, ea_ref, eb_ref, db0_ref, ud_ref, vd_ref, cnt_ref):
        s = p_ref[0] + p_ref[1]
        cnt = s[:, lat:lat + 1]
        z = s[:, 0:lat] / jnp.maximum(cnt, 1.0)
        ud_ref[...] = _dot(z, ea_ref[...]) + db0_ref[...]
        vd_ref[...] = _dot(z, eb_ref[...])
        cnt_ref[...] = cnt

    return pl.pallas_call(
        body,
        grid=(nn // _BN,),
        in_specs=[
            pl.BlockSpec((_NC, _BN, d), lambda i: (0, i, 0)),
            pl.BlockSpec((lat, k), lambda i: (0, 0)),
            pl.BlockSpec((lat, k), lambda i: (0, 0)),
            pl.BlockSpec((1, k), lambda i: (0, 0)),
        ],
        out_specs=[
            pl.BlockSpec((_BN, k), lambda i: (i, 0)),
            pl.BlockSpec((_BN, k), lambda i: (i, 0)),
            pl.BlockSpec((_BN, 1), lambda i: (i, 0)),
        ],
        out_shape=[
            jax.ShapeDtypeStruct((nn, k), jnp.float32),
            jax.ShapeDtypeStruct((nn, k), jnp.float32),
            jax.ShapeDtypeStruct((nn, 1), jnp.float32),
        ],
    )(p, ea, eb, db0)


def _final_tc(pd, cnt, w3, b3):
    nn = pd.shape[1]
    k = pd.shape[2]
    dout = w3.shape[1]

    def body(pd_ref, cnt_ref, w3_ref, b3_ref, out_ref):
        s = pd_ref[0] + pd_ref[1]
        c = cnt_ref[...]
        o = _dot(s / jnp.maximum(c, 1.0), w3_ref[...]) + b3_ref[...]
        out_ref[...] = jnp.where(c > 0.0, o, 0.0)

    return pl.pallas_call(
        body,
        grid=(nn // _BN,),
        in_specs=[
            pl.BlockSpec((_NC, _BN, k), lambda i: (0, i, 0)),
            pl.BlockSpec((_BN, 1), lambda i: (i, 0)),
            pl.BlockSpec((k, dout), lambda i: (0, 0)),
            pl.BlockSpec((1, dout), lambda i: (0, 0)),
        ],
        out_specs=pl.BlockSpec((_BN, dout), lambda i: (i, 0)),
        out_shape=jax.ShapeDtypeStruct((nn, dout), jnp.float32),
    )(pd, cnt, w3, b3)


def _sc_gather_pair(taba, tabb, dstg, srcg):
    nn, k = taba.shape
    ne = dstg.shape[0] * dstg.shape[1]
    ech = ne // _NW
    nch = ech // _C
    ng = nch // _G
    mesh = plsc.VectorSubcoreMesh(core_axis_name="c", subcore_axis_name="s")

    @functools.partial(
        pl.kernel,
        out_type=(jax.ShapeDtypeStruct((ne, k), jnp.float32),
                  jax.ShapeDtypeStruct((ne, k), jnp.float32)),
        mesh=mesh,
        compiler_params=pltpu.CompilerParams(use_tc_tiling_on_sc=False),
        scratch_types=[
            pltpu.VMEM((nch, _C), jnp.int32),
            pltpu.VMEM((nch, _C), jnp.int32),
            pltpu.VMEM((_G, _C, k), jnp.float32),
            pltpu.VMEM((_G, _C, k), jnp.float32),
            pltpu.SemaphoreType.DMA,
            pltpu.SemaphoreType.DMA,
            pltpu.SemaphoreType.DMA,
        ],
    )
    def run(taba_h, tabb_h, dst_h, src_h, outa_h, outb_h,
            idxd, idxs, bufa, bufb, sema, semb, semw):
        wid = lax.axis_index("s") * _NC + lax.axis_index("c")
        cbase = wid * nch
        ebase = wid * ech
        pltpu.sync_copy(dst_h.at[pl.ds(cbase, nch)], idxd)
        pltpu.sync_copy(src_h.at[pl.ds(cbase, nch)], idxs)

        def group(g, carry):
            ha = []
            hb = []
            for b in range(_G):
                j = g * _G + b
                ha.append(pltpu.async_copy(taba_h.at[idxd.at[j]], bufa.at[b],
                                           sema))
                hb.append(pltpu.async_copy(tabb_h.at[idxs.at[j]], bufb.at[b],
                                           semb))
            pend = []
            for b in range(_G):
                j = g * _G + b
                ha[b].wait()
                pend.append(pltpu.async_copy(
                    bufa.at[b], outa_h.at[pl.ds(ebase + j * _C, _C)], semw))
                hb[b].wait()
                pend.append(pltpu.async_copy(
                    bufb.at[b], outb_h.at[pl.ds(ebase + j * _C, _C)], semw))
            for w in pend:
                w.wait()
            return carry

        lax.fori_loop(0, ng, group, 0)

    return run(taba, tabb, dstg, srcg)


def _sc_scatter_add(h, dstg, nn):
    ne, d = h.shape
    ech = ne // _NW
    nch = ech // _C
    rpt = nn // _NS
    mesh = plsc.VectorSubcoreMesh(core_axis_name="c", subcore_axis_name="s")

    @functools.partial(
        pl.kernel,
        out_type=jax.ShapeDtypeStruct((_NC, nn, d), jnp.float32),
        mesh=mesh,
        compiler_params=pltpu.CompilerParams(use_tc_tiling_on_sc=False),
        scratch_types=[
            pltpu.VMEM((nch, _C), jnp.int32),
            pltpu.VMEM((2, _C, d), jnp.float32),
            pltpu.VMEM((rpt, d), jnp.float32),
            pltpu.VMEM_SHARED((nn, d), jnp.float32),
            pltpu.SemaphoreType.DMA,
        ],
    )
    def run(h_h, dst_h, out_h, idxd, hbuf, zbuf, acc, seml):
        cid = lax.axis_index("c")
        sid = lax.axis_index("s")
        wid = sid * _NC + cid
        cbase = wid * nch
        ebase = wid * ech

        zv = jnp.zeros((16,), jnp.float32)

        def zrow(i, carry):
            for t in range(d // 16):
                zbuf[i, pl.ds(t * 16, 16)] = zv
            return carry

        lax.fori_loop(0, rpt, zrow, 0)
        pltpu.sync_copy(zbuf, acc.at[pl.ds(sid * rpt, rpt)])
        pltpu.sync_copy(dst_h.at[pl.ds(cbase, nch)], idxd)
        plsc.subcore_barrier()

        def lstart(j, b):
            return pltpu.async_copy(h_h.at[pl.ds(ebase + j * _C, _C)],
                                    hbuf.at[b], seml)

        def lwait(j, b):
            pltpu.make_async_copy(h_h.at[pl.ds(ebase + j * _C, _C)],
                                  hbuf.at[b], seml).wait()

        def sadd(j, b):
            pltpu.sync_copy(hbuf.at[b], acc.at[idxd.at[j]], add=True)

        lstart(0, 0)

        def pair(g, carry):
            j0 = 2 * g
            lstart(j0 + 1, 1)
            lwait(j0, 0)
            sadd(j0, 0)
            lstart(j0 + 2, 0)
            lwait(j0 + 1, 1)
            sadd(j0 + 1, 1)
            return carry

        lax.fori_loop(0, (nch - 1) // 2, pair, 0)
        lwait(nch - 1, 0)
        sadd(nch - 1, 0)

        plsc.subcore_barrier()
        pltpu.sync_copy(acc.at[pl.ds(sid * rpt, rpt)],
                        out_h.at[cid, pl.ds(sid * rpt, rpt)])

    return run(h, dstg)


def kernel(x, edge_index,
           enc_W0, enc_b0, enc_W1, enc_b1, enc_W2, enc_b2, enc_W3, enc_b3,
           dec_W0, dec_b0, dec_W1, dec_b1, dec_W2, dec_b2, dec_W3, dec_b3):
    nn, din = x.shape
    ne = edge_index.shape[1]
    lat = enc_W3.shape[1]

    src = edge_index[0]
    dst = edge_index[1]
    dstg = dst.reshape(ne // _C, _C)
    srcg = src.reshape(ne // _C, _C)

    f32 = jnp.float32
    wa = enc_W0[:din] - enc_W0[din:]
    wb = enc_W0[din:]
    b0 = enc_b0.reshape(1, -1)
    b1 = enc_b1.reshape(1, -1)
    b2 = enc_b2.reshape(1, -1)
    w3p = jnp.concatenate([enc_W3, jnp.zeros((enc_W3.shape[0], 16 - lat), f32)],
                          axis=1)
    b3p = jnp.concatenate(
        [enc_b3, jnp.ones((1,), f32), jnp.zeros((16 - lat - 1,), f32)]
    ).reshape(1, 16)

    ea = dec_W0[:lat] - dec_W0[lat:]
    eb = dec_W0[lat:]
    db0 = dec_b0.reshape(1, -1)
    db1 = dec_b1.reshape(1, -1)
    db2 = dec_b2.reshape(1, -1)
    db3 = dec_b3.reshape(1, -1)

    u, v = _node_tables_tc(x, wa, wb, b0)
    gu, gv = _sc_gather_pair(u, v, dstg, srcg)
    e4 = _edge_mlp_tc(gu, gv, enc_W1, b1, enc_W2, b2, w3p, b3p)
    p = _sc_scatter_add(e4, dstg, nn)
    ud, vd, cnt = _dec_tables_tc(p, ea, eb, db0)
    gud, gvd = _sc_gather_pair(ud, vd, dstg, srcg)
    h3 = _edge_mlp_tc(gud, gvd, dec_W1, db1, dec_W2, db2)
    pd = _sc_scatter_add(h3, dstg, nn)
    return _final_tc(pd, cnt, dec_W3, db3)


# speedup vs baseline: 5.8398x; 2.1032x over previous
import functools

import jax
import jax.numpy as jnp
from jax import lax
from jax.experimental import pallas as pl
from jax.experimental.pallas import tpu as pltpu
from jax.experimental.pallas import tpu_sc as plsc

_NC = 2
_NS = 16
_NW = _NC * _NS
_C = 80
_G = 5

_BN = 2000
_BE = 8000


def _dot(a, b):
    return lax.dot_general(a, b, (((1,), (0,)), ((), ())),
                           precision=lax.Precision.HIGHEST,
                           preferred_element_type=jnp.float32)


def _node_tables_tc(x, wa, wb, b0):
    nn, d = x.shape
    k = wa.shape[1]

    def body(x_ref, wa_ref, wb_ref, b0_ref, u_ref, v_ref):
        xb = x_ref[...]
        u_ref[...] = _dot(xb, wa_ref[...]) + b0_ref[...]
        v_ref[...] = _dot(xb, wb_ref[...])

    return pl.pallas_call(
        body,
        grid=(nn // _BN,),
        in_specs=[
            pl.BlockSpec((_BN, d), lambda i: (i, 0)),
            pl.BlockSpec((d, k), lambda i: (0, 0)),
            pl.BlockSpec((d, k), lambda i: (0, 0)),
            pl.BlockSpec((1, k), lambda i: (0, 0)),
        ],
        out_specs=[
            pl.BlockSpec((_BN, k), lambda i: (i, 0)),
            pl.BlockSpec((_BN, k), lambda i: (i, 0)),
        ],
        out_shape=[
            jax.ShapeDtypeStruct((nn, k), jnp.float32),
            jax.ShapeDtypeStruct((nn, k), jnp.float32),
        ],
    )(x, wa, wb, b0)


_PK = 8


def _blkdiag(w):
    fi, fo = w.shape
    out = jnp.zeros((_PK * fi, _PK * fo), jnp.float32)
    for t in range(_PK):
        out = lax.dynamic_update_slice(out, w, (t * fi, t * fo))
    return out


def _edge_mlp_tc(gu, gv, w1, b1, w2, b2, w3=None, b3=None):
    nr, k = gu.shape
    k1 = w1.shape[1]
    k2 = w2.shape[1]
    kout = w3.shape[1] if w3 is not None else k2
    three = w3 is not None
    br = _BE // _PK

    def body(gu_ref, gv_ref, w1_ref, b1_ref, w2_ref, b2_ref, *rest):
        out_ref = rest[-1]
        h = jnp.maximum(gu_ref[...] + gv_ref[...], 0.0)
        h = jnp.maximum(_dot(h, w1_ref[...]) + b1_ref[...], 0.0)
        h = jnp.maximum(_dot(h, w2_ref[...]) + b2_ref[...], 0.0)
        if three:
            h = jnp.maximum(_dot(h, rest[0][...]) + rest[1][...], 0.0)
        out_ref[...] = h

    in_specs = [
        pl.BlockSpec((br, k), lambda i: (i, 0)),
        pl.BlockSpec((br, k), lambda i: (i, 0)),
        pl.BlockSpec((k, k1), lambda i: (0, 0)),
        pl.BlockSpec((1, k1), lambda i: (0, 0)),
        pl.BlockSpec((k1, k2), lambda i: (0, 0)),
        pl.BlockSpec((1, k2), lambda i: (0, 0)),
    ]
    args = [gu, gv, w1, b1, w2, b2]
    if three:
        in_specs += [pl.BlockSpec((k2, kout), lambda i: (0, 0)),
                     pl.BlockSpec((1, kout), lambda i: (0, 0))]
        args += [w3, b3]
    return pl.pallas_call(
        body,
        grid=(nr // br,),
        in_specs=in_specs,
        out_specs=pl.BlockSpec((br, kout), lambda i: (i, 0)),
        out_shape=jax.ShapeDtypeStruct((nr, kout), jnp.float32),
    )(*args)


def _dec_tables_tc(p, ea, eb, db0):
    nn = p.shape[1]
    lat = ea.shape[0]
    k = ea.shape[1]
    d = p.shape[2]

    def body(p_ref, ea_ref, eb_ref, db0_ref, ud_ref, vd_ref, cnt_ref):
        s = p_ref[0] + p_ref[1]
        cnt = s[:, lat:lat + 1]
        z = s[:, 0:lat] / jnp.maximum(cnt, 1.0)
        ud_ref[...] = _dot(z, ea_ref[...]) + db0_ref[...]
        vd_ref[...] = _dot(z, eb_ref[...])
        cnt_ref[...] = cnt

    return pl.pallas_call(
        body,
        grid=(nn // _BN,),
        in_specs=[
            pl.BlockSpec((_NC, _BN, d), lambda i: (0, i, 0)),
            pl.BlockSpec((lat, k), lambda i: (0, 0)),
            pl.BlockSpec((lat, k), lambda i: (0, 0)),
            pl.BlockSpec((1, k), lambda i: (0, 0)),
        ],
        out_specs=[
            pl.BlockSpec((_BN, k), lambda i: (i, 0)),
            pl.BlockSpec((_BN, k), lambda i: (i, 0)),
            pl.BlockSpec((_BN, 1), lambda i: (i, 0)),
        ],
        out_shape=[
            jax.ShapeDtypeStruct((nn, k), jnp.float32),
            jax.ShapeDtypeStruct((nn, k), jnp.float32),
            jax.ShapeDtypeStruct((nn, 1), jnp.float32),
        ],
    )(p, ea, eb, db0)


def _final_tc(pd, cnt, w3, b3):
    nn = pd.shape[1]
    k = pd.shape[2]
    dout = w3.shape[1]

    def body(pd_ref, cnt_ref, w3_ref, b3_ref, out_ref):
        s = pd_ref[0] + pd_ref[1]
        c = cnt_ref[...]
        o = _dot(s / jnp.maximum(c, 1.0), w3_ref[...]) + b3_ref[...]
        out_ref[...] = jnp.where(c > 0.0, o, 0.0)

    return pl.pallas_call(
        body,
        grid=(nn // _BN,),
        in_specs=[
            pl.BlockSpec((_NC, _BN, k), lambda i: (0, i, 0)),
            pl.BlockSpec((_BN, 1), lambda i: (i, 0)),
            pl.BlockSpec((k, dout), lambda i: (0, 0)),
            pl.BlockSpec((1, dout), lambda i: (0, 0)),
        ],
        out_specs=pl.BlockSpec((_BN, dout), lambda i: (i, 0)),
        out_shape=jax.ShapeDtypeStruct((nn, dout), jnp.float32),
    )(pd, cnt, w3, b3)


def _sc_gather_pair(taba, tabb, dstg, srcg):
    nn, k = taba.shape
    ne = dstg.shape[0] * dstg.shape[1]
    ech = ne // _NW
    nch = ech // _C
    ng = nch // _G
    mesh = plsc.VectorSubcoreMesh(core_axis_name="c", subcore_axis_name="s")

    @functools.partial(
        pl.kernel,
        out_type=(jax.ShapeDtypeStruct((ne, k), jnp.float32),
                  jax.ShapeDtypeStruct((ne, k), jnp.float32)),
        mesh=mesh,
        compiler_params=pltpu.CompilerParams(use_tc_tiling_on_sc=False),
        scratch_types=[
            pltpu.VMEM((nch, _C), jnp.int32),
            pltpu.VMEM((nch, _C), jnp.int32),
            pltpu.VMEM((_G, _C, k), jnp.float32),
            pltpu.VMEM((_G, _C, k), jnp.float32),
            pltpu.SemaphoreType.DMA,
            pltpu.SemaphoreType.DMA,
            pltpu.SemaphoreType.DMA,
        ],
    )
    def run(taba_h, tabb_h, dst_h, src_h, outa_h, outb_h,
            idxd, idxs, bufa, bufb, sema, semb, semw):
        wid = lax.axis_index("s") * _NC + lax.axis_index("c")
        cbase = wid * nch
        ebase = wid * ech
        pltpu.sync_copy(dst_h.at[pl.ds(cbase, nch)], idxd)
        pltpu.sync_copy(src_h.at[pl.ds(cbase, nch)], idxs)

        def group(g, carry):
            ha = []
            hb = []
            for b in range(_G):
                j = g * _G + b
                ha.append(pltpu.async_copy(taba_h.at[idxd.at[j]], bufa.at[b],
                                           sema))
                hb.append(pltpu.async_copy(tabb_h.at[idxs.at[j]], bufb.at[b],
                                           semb))
            pend = []
            for b in range(_G):
                j = g * _G + b
                ha[b].wait()
                pend.append(pltpu.async_copy(
                    bufa.at[b], outa_h.at[pl.ds(ebase + j * _C, _C)], semw))
                hb[b].wait()
                pend.append(pltpu.async_copy(
                    bufb.at[b], outb_h.at[pl.ds(ebase + j * _C, _C)], semw))
            for w in pend:
                w.wait()
            return carry

        lax.fori_loop(0, ng, group, 0)

    return run(taba, tabb, dstg, srcg)


def _sc_scatter_add(h, dstg, nn):
    ne, d = h.shape
    ech = ne // _NW
    nch = ech // _C
    rpt = nn // _NS
    mesh = plsc.VectorSubcoreMesh(core_axis_name="c", subcore_axis_name="s")

    @functools.partial(
        pl.kernel,
        out_type=jax.ShapeDtypeStruct((_NC, nn, d), jnp.float32),
        mesh=mesh,
        compiler_params=pltpu.CompilerParams(use_tc_tiling_on_sc=False),
        scratch_types=[
            pltpu.VMEM((nch, _C), jnp.int32),
            pltpu.VMEM((2, _C, d), jnp.float32),
            pltpu.VMEM((rpt, d), jnp.float32),
            pltpu.VMEM_SHARED((nn, d), jnp.float32),
            pltpu.SemaphoreType.DMA,
        ],
    )
    def run(h_h, dst_h, out_h, idxd, hbuf, zbuf, acc, seml):
        cid = lax.axis_index("c")
        sid = lax.axis_index("s")
        wid = sid * _NC + cid
        cbase = wid * nch
        ebase = wid * ech

        zv = jnp.zeros((16,), jnp.float32)

        def zrow(i, carry):
            for t in range(d // 16):
                zbuf[i, pl.ds(t * 16, 16)] = zv
            return carry

        lax.fori_loop(0, rpt, zrow, 0)
        pltpu.sync_copy(zbuf, acc.at[pl.ds(sid * rpt, rpt)])
        pltpu.sync_copy(dst_h.at[pl.ds(cbase, nch)], idxd)
        plsc.subcore_barrier()

        def lstart(j, b):
            return pltpu.async_copy(h_h.at[pl.ds(ebase + j * _C, _C)],
                                    hbuf.at[b], seml)

        def lwait(j, b):
            pltpu.make_async_copy(h_h.at[pl.ds(ebase + j * _C, _C)],
                                  hbuf.at[b], seml).wait()

        def sadd(j, b):
            pltpu.sync_copy(hbuf.at[b], acc.at[idxd.at[j]], add=True)

        lstart(0, 0)

        def pair(g, carry):
            j0 = 2 * g
            lstart(j0 + 1, 1)
            lwait(j0, 0)
            sadd(j0, 0)
            lstart(j0 + 2, 0)
            lwait(j0 + 1, 1)
            sadd(j0 + 1, 1)
            return carry

        lax.fori_loop(0, (nch - 1) // 2, pair, 0)
        lwait(nch - 1, 0)
        sadd(nch - 1, 0)

        plsc.subcore_barrier()
        pltpu.sync_copy(acc.at[pl.ds(sid * rpt, rpt)],
                        out_h.at[cid, pl.ds(sid * rpt, rpt)])

    return run(h, dstg)


def kernel(x, edge_index,
           enc_W0, enc_b0, enc_W1, enc_b1, enc_W2, enc_b2, enc_W3, enc_b3,
           dec_W0, dec_b0, dec_W1, dec_b1, dec_W2, dec_b2, dec_W3, dec_b3):
    nn, din = x.shape
    ne = edge_index.shape[1]
    lat = enc_W3.shape[1]

    src = edge_index[0]
    dst = edge_index[1]
    dstg = dst.reshape(ne // _C, _C)
    srcg = src.reshape(ne // _C, _C)

    f32 = jnp.float32
    wa = enc_W0[:din] - enc_W0[din:]
    wb = enc_W0[din:]
    b0 = enc_b0.reshape(1, -1)
    b1 = enc_b1.reshape(1, -1)
    b2 = enc_b2.reshape(1, -1)
    w3p = jnp.concatenate([enc_W3, jnp.zeros((enc_W3.shape[0], 16 - lat), f32)],
                          axis=1)
    b3p = jnp.concatenate(
        [enc_b3, jnp.ones((1,), f32), jnp.zeros((16 - lat - 1,), f32)]
    ).reshape(1, 16)

    ea = dec_W0[:lat] - dec_W0[lat:]
    eb = dec_W0[lat:]
    db0 = dec_b0.reshape(1, -1)
    db1 = dec_b1.reshape(1, -1)
    db2 = dec_b2.reshape(1, -1)
    db3 = dec_b3.reshape(1, -1)

    w1b, w2b, w3pb = _blkdiag(enc_W1), _blkdiag(enc_W2), _blkdiag(w3p)
    b1b, b2b, b3pb = (jnp.tile(b1, (1, _PK)), jnp.tile(b2, (1, _PK)),
                      jnp.tile(b3p, (1, _PK)))
    dw1b, dw2b = _blkdiag(dec_W1), _blkdiag(dec_W2)
    db1b, db2b = jnp.tile(db1, (1, _PK)), jnp.tile(db2, (1, _PK))

    u, v = _node_tables_tc(x, wa, wb, b0)
    gu, gv = _sc_gather_pair(u, v, dstg, srcg)
    gu8 = gu.reshape(ne // _PK, _PK * 32)
    gv8 = gv.reshape(ne // _PK, _PK * 32)
    e4 = _edge_mlp_tc(gu8, gv8, w1b, b1b, w2b, b2b, w3pb, b3pb)
    p = _sc_scatter_add(e4.reshape(ne, 16), dstg, nn)
    ud, vd, cnt = _dec_tables_tc(p, ea, eb, db0)
    gud, gvd = _sc_gather_pair(ud, vd, dstg, srcg)
    gud8 = gud.reshape(ne // _PK, _PK * 32)
    gvd8 = gvd.reshape(ne // _PK, _PK * 32)
    h3 = _edge_mlp_tc(gud8, gvd8, dw1b, db1b, dw2b, db2b)
    pd = _sc_scatter_add(h3.reshape(ne, 32), dstg, nn)
    return _final_tc(pd, cnt, dec_W3, db3)


# speedup vs baseline: 6.7572x; 1.1571x over previous
import functools

import jax
import jax.numpy as jnp
from jax import lax
from jax.experimental import pallas as pl
from jax.experimental.pallas import tpu as pltpu
from jax.experimental.pallas import tpu_sc as plsc

_NC = 2
_NS = 16
_NW = _NC * _NS
_C = 80
_G = 5

_BN = 2000
_BE = 8000


def _dot(a, b, prec=lax.Precision.HIGHEST):
    return lax.dot_general(a, b, (((1,), (0,)), ((), ())),
                           precision=prec,
                           preferred_element_type=jnp.float32)


def _node_tables_tc(x, wa, wb, b0):
    nn, d = x.shape
    k = wa.shape[1]

    def body(x_ref, wa_ref, wb_ref, b0_ref, u_ref, v_ref):
        xb = x_ref[...]
        u_ref[...] = _dot(xb, wa_ref[...]) + b0_ref[...]
        v_ref[...] = _dot(xb, wb_ref[...])

    return pl.pallas_call(
        body,
        grid=(nn // _BN,),
        in_specs=[
            pl.BlockSpec((_BN, d), lambda i: (i, 0)),
            pl.BlockSpec((d, k), lambda i: (0, 0)),
            pl.BlockSpec((d, k), lambda i: (0, 0)),
            pl.BlockSpec((1, k), lambda i: (0, 0)),
        ],
        out_specs=[
            pl.BlockSpec((_BN, k), lambda i: (i, 0)),
            pl.BlockSpec((_BN, k), lambda i: (i, 0)),
        ],
        out_shape=[
            jax.ShapeDtypeStruct((nn, k), jnp.float32),
            jax.ShapeDtypeStruct((nn, k), jnp.float32),
        ],
    )(x, wa, wb, b0)


_PK = 8


def _blkdiag(w):
    fi, fo = w.shape
    out = jnp.zeros((_PK * fi, _PK * fo), jnp.float32)
    for t in range(_PK):
        out = lax.dynamic_update_slice(out, w, (t * fi, t * fo))
    return out


def _edge_mlp_tc(gu, gv, w1, b1, w2, b2, w3=None, b3=None,
                 prec=lax.Precision.HIGHEST):
    nr, k = gu.shape
    k1 = w1.shape[1]
    k2 = w2.shape[1]
    kout = w3.shape[1] if w3 is not None else k2
    three = w3 is not None
    br = _BE // _PK

    def body(gu_ref, gv_ref, w1_ref, b1_ref, w2_ref, b2_ref, *rest):
        out_ref = rest[-1]
        h = jnp.maximum(gu_ref[...] + gv_ref[...], 0.0)
        h = jnp.maximum(_dot(h, w1_ref[...], prec) + b1_ref[...], 0.0)
        h = jnp.maximum(_dot(h, w2_ref[...], prec) + b2_ref[...], 0.0)
        if three:
            h = jnp.maximum(_dot(h, rest[0][...], prec) + rest[1][...], 0.0)
        out_ref[...] = h

    in_specs = [
        pl.BlockSpec((br, k), lambda i: (i, 0)),
        pl.BlockSpec((br, k), lambda i: (i, 0)),
        pl.BlockSpec((k, k1), lambda i: (0, 0)),
        pl.BlockSpec((1, k1), lambda i: (0, 0)),
        pl.BlockSpec((k1, k2), lambda i: (0, 0)),
        pl.BlockSpec((1, k2), lambda i: (0, 0)),
    ]
    args = [gu, gv, w1, b1, w2, b2]
    if three:
        in_specs += [pl.BlockSpec((k2, kout), lambda i: (0, 0)),
                     pl.BlockSpec((1, kout), lambda i: (0, 0))]
        args += [w3, b3]
    return pl.pallas_call(
        body,
        grid=(nr // br,),
        in_specs=in_specs,
        out_specs=pl.BlockSpec((br, kout), lambda i: (i, 0)),
        out_shape=jax.ShapeDtypeStruct((nr, kout), jnp.float32),
    )(*args)


def _dec_tables_tc(p, ea, eb, db0):
    nn = p.shape[1]
    lat = ea.shape[0]
    k = ea.shape[1]
    d = p.shape[2]

    def body(p_ref, ea_ref, eb_ref, db0_ref, ud_ref, vd_ref, cnt_ref):
        s = p_ref[0] + p_ref[1]
        cnt = s[:, lat:lat + 1]
        z = s[:, 0:lat] / jnp.maximum(cnt, 1.0)
        ud_ref[...] = _dot(z, ea_ref[...]) + db0_ref[...]
        vd_ref[...] = _dot(z, eb_ref[...])
        cnt_ref[...] = cnt

    return pl.pallas_call(
        body,
        grid=(nn // _BN,),
        in_specs=[
            pl.BlockSpec((_NC, _BN, d), lambda i: (0, i, 0)),
            pl.BlockSpec((lat, k), lambda i: (0, 0)),
            pl.BlockSpec((lat, k), lambda i: (0, 0)),
            pl.BlockSpec((1, k), lambda i: (0, 0)),
        ],
        out_specs=[
            pl.BlockSpec((_BN, k), lambda i: (i, 0)),
            pl.BlockSpec((_BN, k), lambda i: (i, 0)),
            pl.BlockSpec((_BN, 1), lambda i: (i, 0)),
        ],
        out_shape=[
            jax.ShapeDtypeStruct((nn, k), jnp.float32),
            jax.ShapeDtypeStruct((nn, k), jnp.float32),
            jax.ShapeDtypeStruct((nn, 1), jnp.float32),
        ],
    )(p, ea, eb, db0)


def _final_tc(pd, cnt, w3, b3):
    nn = pd.shape[1]
    k = pd.shape[2]
    dout = w3.shape[1]

    def body(pd_ref, cnt_ref, w3_ref, b3_ref, out_ref):
        s = pd_ref[0] + pd_ref[1]
        c = cnt_ref[...]
        o = _dot(s / jnp.maximum(c, 1.0), w3_ref[...]) + b3_ref[...]
        out_ref[...] = jnp.where(c > 0.0, o, 0.0)

    return pl.pallas_call(
        body,
        grid=(nn // _BN,),
        in_specs=[
            pl.BlockSpec((_NC, _BN, k), lambda i: (0, i, 0)),
            pl.BlockSpec((_BN, 1), lambda i: (i, 0)),
            pl.BlockSpec((k, dout), lambda i: (0, 0)),
            pl.BlockSpec((1, dout), lambda i: (0, 0)),
        ],
        out_specs=pl.BlockSpec((_BN, dout), lambda i: (i, 0)),
        out_shape=jax.ShapeDtypeStruct((nn, dout), jnp.float32),
    )(pd, cnt, w3, b3)


def _sc_gather_pair(taba, tabb, dstg, srcg):
    nn, k = taba.shape
    ne = dstg.shape[0] * dstg.shape[1]
    ech = ne // _NW
    nch = ech // _C
    ng = nch // _G
    kp = _PK * k
    rch = _C // _PK
    mesh = plsc.VectorSubcoreMesh(core_axis_name="c", subcore_axis_name="s")

    @functools.partial(
        pl.kernel,
        out_type=(jax.ShapeDtypeStruct((ne, k), jnp.float32),
                  jax.ShapeDtypeStruct((ne, k), jnp.float32)),
        mesh=mesh,
        compiler_params=pltpu.CompilerParams(use_tc_tiling_on_sc=False),
        scratch_types=[
            pltpu.VMEM((nch, _C), jnp.int32),
            pltpu.VMEM((nch, _C), jnp.int32),
            pltpu.VMEM((_G, _C, k), jnp.float32),
            pltpu.VMEM((_G, _C, k), jnp.float32),
            pltpu.SemaphoreType.DMA,
            pltpu.SemaphoreType.DMA,
            pltpu.SemaphoreType.DMA,
        ],
    )
    def run(taba_h, tabb_h, dst_h, src_h, outa_h, outb_h,
            idxd, idxs, bufa, bufb, sema, semb, semw):
        wid = lax.axis_index("s") * _NC + lax.axis_index("c")
        cbase = wid * nch
        ebase = wid * ech
        pltpu.sync_copy(dst_h.at[pl.ds(cbase, nch)], idxd)
        pltpu.sync_copy(src_h.at[pl.ds(cbase, nch)], idxs)

        def group(g, carry):
            ha = []
            hb = []
            for b in range(_G):
                j = g * _G + b
                ha.append(pltpu.async_copy(taba_h.at[idxd.at[j]], bufa.at[b],
                                           sema))
                hb.append(pltpu.async_copy(tabb_h.at[idxs.at[j]], bufb.at[b],
                                           semb))
            pend = []
            for b in range(_G):
                j = g * _G + b
                ha[b].wait()
                pend.append(pltpu.async_copy(
                    bufa.at[b],
                    outa_h.at[pl.ds(ebase + j * _C, _C)], semw))
                hb[b].wait()
                pend.append(pltpu.async_copy(
                    bufb.at[b],
                    outb_h.at[pl.ds(ebase + j * _C, _C)], semw))
            for w in pend:
                w.wait()
            return carry

        lax.fori_loop(0, ng, group, 0)

    return run(taba, tabb, dstg, srcg)


def _sc_scatter_add(h, dstg, nn):
    ne, d = h.shape
    ech = ne // _NW
    nch = ech // _C
    rpt = nn // _NS
    mesh = plsc.VectorSubcoreMesh(core_axis_name="c", subcore_axis_name="s")

    @functools.partial(
        pl.kernel,
        out_type=jax.ShapeDtypeStruct((_NC, nn, d), jnp.float32),
        mesh=mesh,
        compiler_params=pltpu.CompilerParams(use_tc_tiling_on_sc=False),
        scratch_types=[
            pltpu.VMEM((nch, _C), jnp.int32),
            pltpu.VMEM((2, _C, d), jnp.float32),
            pltpu.VMEM((rpt, d), jnp.float32),
            pltpu.VMEM_SHARED((nn, d), jnp.float32),
            pltpu.SemaphoreType.DMA,
        ],
    )
    def run(h_h, dst_h, out_h, idxd, hbuf, zbuf, acc, seml):
        cid = lax.axis_index("c")
        sid = lax.axis_index("s")
        wid = sid * _NC + cid
        cbase = wid * nch
        ebase = wid * ech

        zv = jnp.zeros((16,), jnp.float32)

        def zrow(i, carry):
            for t in range(d // 16):
                zbuf[i, pl.ds(t * 16, 16)] = zv
            return carry

        lax.fori_loop(0, rpt, zrow, 0)
        pltpu.sync_copy(zbuf, acc.at[pl.ds(sid * rpt, rpt)])
        pltpu.sync_copy(dst_h.at[pl.ds(cbase, nch)], idxd)
        plsc.subcore_barrier()

        def lstart(j, b):
            return pltpu.async_copy(h_h.at[pl.ds(ebase + j * _C, _C)],
                                    hbuf.at[b], seml)

        def lwait(j, b):
            pltpu.make_async_copy(h_h.at[pl.ds(ebase + j * _C, _C)],
                                  hbuf.at[b], seml).wait()

        def sadd(j, b):
            pltpu.sync_copy(hbuf.at[b], acc.at[idxd.at[j]], add=True)

        lstart(0, 0)

        def pair(g, carry):
            j0 = 2 * g
            lstart(j0 + 1, 1)
            lwait(j0, 0)
            sadd(j0, 0)
            lstart(j0 + 2, 0)
            lwait(j0 + 1, 1)
            sadd(j0 + 1, 1)
            return carry

        lax.fori_loop(0, (nch - 1) // 2, pair, 0)
        lwait(nch - 1, 0)
        sadd(nch - 1, 0)

        plsc.subcore_barrier()
        pltpu.sync_copy(acc.at[pl.ds(sid * rpt, rpt)],
                        out_h.at[cid, pl.ds(sid * rpt, rpt)])

    return run(h, dstg)


def kernel(x, edge_index,
           enc_W0, enc_b0, enc_W1, enc_b1, enc_W2, enc_b2, enc_W3, enc_b3,
           dec_W0, dec_b0, dec_W1, dec_b1, dec_W2, dec_b2, dec_W3, dec_b3):
    nn, din = x.shape
    ne = edge_index.shape[1]
    lat = enc_W3.shape[1]

    src = edge_index[0]
    dst = edge_index[1]
    dstg = dst.reshape(ne // _C, _C)
    srcg = src.reshape(ne // _C, _C)

    f32 = jnp.float32
    wa = enc_W0[:din] - enc_W0[din:]
    wb = enc_W0[din:]
    b0 = enc_b0.reshape(1, -1)
    b1 = enc_b1.reshape(1, -1)
    b2 = enc_b2.reshape(1, -1)
    w3p = jnp.concatenate([enc_W3, jnp.zeros((enc_W3.shape[0], 16 - lat), f32)],
                          axis=1)
    b3p = jnp.concatenate(
        [enc_b3, jnp.ones((1,), f32), jnp.zeros((16 - lat - 1,), f32)]
    ).reshape(1, 16)

    ea = dec_W0[:lat] - dec_W0[lat:]
    eb = dec_W0[lat:]
    db0 = dec_b0.reshape(1, -1)
    db1 = dec_b1.reshape(1, -1)
    db2 = dec_b2.reshape(1, -1)
    db3 = dec_b3.reshape(1, -1)

    w1b, w2b, w3pb = _blkdiag(enc_W1), _blkdiag(enc_W2), _blkdiag(w3p)
    b1b, b2b, b3pb = (jnp.tile(b1, (1, _PK)), jnp.tile(b2, (1, _PK)),
                      jnp.tile(b3p, (1, _PK)))
    dw1b, dw2b = _blkdiag(dec_W1), _blkdiag(dec_W2)
    db1b, db2b = jnp.tile(db1, (1, _PK)), jnp.tile(db2, (1, _PK))

    u, v = _node_tables_tc(x, wa, wb, b0)
    gu, gv = _sc_gather_pair(u, v, dstg, srcg)
    gu8 = gu.reshape(ne // _PK, _PK * 32)
    gv8 = gv.reshape(ne // _PK, _PK * 32)
    e4 = _edge_mlp_tc(gu8, gv8, w1b, b1b, w2b, b2b, w3pb, b3pb)
    p = _sc_scatter_add(e4.reshape(ne, 16), dstg, nn)
    ud, vd, cnt = _dec_tables_tc(p, ea, eb, db0)
    gud, gvd = _sc_gather_pair(ud, vd, dstg, srcg)
    gud8 = gud.reshape(ne // _PK, _PK * 32)
    gvd8 = gvd.reshape(ne // _PK, _PK * 32)
    h3 = _edge_mlp_tc(gud8, gvd8, dw1b, db1b, dw2b, db2b,
                      prec=lax.Precision.DEFAULT)
    pd = _sc_scatter_add(h3.reshape(ne, 32), dstg, nn)
    return _final_tc(pd, cnt, dec_W3, db3)


# speedup vs baseline: 7.5924x; 1.1236x over previous
import functools

import jax
import jax.numpy as jnp
from jax import lax
from jax.experimental import pallas as pl
from jax.experimental.pallas import tpu as pltpu
from jax.experimental.pallas import tpu_sc as plsc

_NC = 2
_NS = 16
_NW = _NC * _NS
_C = 80
_G = 5

_BN = 2000
_BE = 8000


def _dot(a, b, prec=lax.Precision.HIGHEST):
    return lax.dot_general(a, b, (((1,), (0,)), ((), ())),
                           precision=prec,
                           preferred_element_type=jnp.float32)


def _node_tables_tc(x, wa, wb, b0):
    nn, d = x.shape
    k = wa.shape[1]

    def body(x_ref, wa_ref, wb_ref, b0_ref, u_ref, v_ref):
        xb = x_ref[...]
        u_ref[...] = _dot(xb, wa_ref[...]) + b0_ref[...]
        v_ref[...] = _dot(xb, wb_ref[...])

    return pl.pallas_call(
        body,
        grid=(nn // _BN,),
        in_specs=[
            pl.BlockSpec((_BN, d), lambda i: (i, 0)),
            pl.BlockSpec((d, k), lambda i: (0, 0)),
            pl.BlockSpec((d, k), lambda i: (0, 0)),
            pl.BlockSpec((1, k), lambda i: (0, 0)),
        ],
        out_specs=[
            pl.BlockSpec((_BN, k), lambda i: (i, 0)),
            pl.BlockSpec((_BN, k), lambda i: (i, 0)),
        ],
        out_shape=[
            jax.ShapeDtypeStruct((nn, k), jnp.float32),
            jax.ShapeDtypeStruct((nn, k), jnp.float32),
        ],
    )(x, wa, wb, b0)


_PK = 8


def _blkdiag(w):
    fi, fo = w.shape
    out = jnp.zeros((_PK * fi, _PK * fo), jnp.float32)
    for t in range(_PK):
        out = lax.dynamic_update_slice(out, w, (t * fi, t * fo))
    return out


def _split_bf16(w):
    hi = w.astype(jnp.bfloat16)
    lo = (w - hi.astype(jnp.float32)).astype(jnp.bfloat16)
    return hi, lo


def _edge_mlp_x3_tc(gu, gv, ws, bs):
    nr, k = gu.shape
    br = _BE // _PK
    nl = len(ws)

    def body(*refs):
        gu_ref, gv_ref = refs[0], refs[1]
        out_ref = refs[-1]
        h = jnp.maximum(gu_ref[...] + gv_ref[...], 0.0)
        for i in range(nl):
            whi = refs[2 + 3 * i][...]
            wlo = refs[3 + 3 * i][...]
            b = refs[4 + 3 * i][...]
            hhi = h.astype(jnp.bfloat16)
            hlo = (h - hhi.astype(jnp.float32)).astype(jnp.bfloat16)
            acc = _dot(hhi, whi, lax.Precision.DEFAULT)
            acc = acc + _dot(hlo, whi, lax.Precision.DEFAULT)
            acc = acc + _dot(hhi, wlo, lax.Precision.DEFAULT)
            h = jnp.maximum(acc + b, 0.0)
        out_ref[...] = h

    in_specs = [
        pl.BlockSpec((br, k), lambda i: (i, 0)),
        pl.BlockSpec((br, k), lambda i: (i, 0)),
    ]
    args = [gu, gv]
    for (whi, wlo), b in zip(ws, bs):
        ki, ko = whi.shape
        in_specs += [pl.BlockSpec((ki, ko), lambda i: (0, 0)),
                     pl.BlockSpec((ki, ko), lambda i: (0, 0)),
                     pl.BlockSpec((1, ko), lambda i: (0, 0))]
        args += [whi, wlo, b]
    kout = ws[-1][0].shape[1]
    return pl.pallas_call(
        body,
        grid=(nr // br,),
        in_specs=in_specs,
        out_specs=pl.BlockSpec((br, kout), lambda i: (i, 0)),
        out_shape=jax.ShapeDtypeStruct((nr, kout), jnp.float32),
    )(*args)


def _edge_mlp_tc(gu, gv, w1, b1, w2, b2, w3=None, b3=None,
                 prec=lax.Precision.HIGHEST):
    nr, k = gu.shape
    k1 = w1.shape[1]
    k2 = w2.shape[1]
    kout = w3.shape[1] if w3 is not None else k2
    three = w3 is not None
    br = _BE // _PK

    def body(gu_ref, gv_ref, w1_ref, b1_ref, w2_ref, b2_ref, *rest):
        out_ref = rest[-1]
        h = jnp.maximum(gu_ref[...] + gv_ref[...], 0.0)
        h = jnp.maximum(_dot(h, w1_ref[...], prec) + b1_ref[...], 0.0)
        h = jnp.maximum(_dot(h, w2_ref[...], prec) + b2_ref[...], 0.0)
        if three:
            h = jnp.maximum(_dot(h, rest[0][...], prec) + rest[1][...], 0.0)
        out_ref[...] = h

    in_specs = [
        pl.BlockSpec((br, k), lambda i: (i, 0)),
        pl.BlockSpec((br, k), lambda i: (i, 0)),
        pl.BlockSpec((k, k1), lambda i: (0, 0)),
        pl.BlockSpec((1, k1), lambda i: (0, 0)),
        pl.BlockSpec((k1, k2), lambda i: (0, 0)),
        pl.BlockSpec((1, k2), lambda i: (0, 0)),
    ]
    args = [gu, gv, w1, b1, w2, b2]
    if three:
        in_specs += [pl.BlockSpec((k2, kout), lambda i: (0, 0)),
                     pl.BlockSpec((1, kout), lambda i: (0, 0))]
        args += [w3, b3]
    return pl.pallas_call(
        body,
        grid=(nr // br,),
        in_specs=in_specs,
        out_specs=pl.BlockSpec((br, kout), lambda i: (i, 0)),
        out_shape=jax.ShapeDtypeStruct((nr, kout), jnp.float32),
    )(*args)


def _dec_tables_tc(p, ea, eb, db0):
    nn = p.shape[1]
    lat = ea.shape[0]
    k = ea.shape[1]
    d = p.shape[2]

    def body(p_ref, ea_ref, eb_ref, db0_ref, ud_ref, vd_ref, cnt_ref):
        s = p_ref[0] + p_ref[1]
        cnt = s[:, lat:lat + 1]
        z = s[:, 0:lat] / jnp.maximum(cnt, 1.0)
        ud_ref[...] = _dot(z, ea_ref[...]) + db0_ref[...]
        vd_ref[...] = _dot(z, eb_ref[...])
        cnt_ref[...] = cnt

    return pl.pallas_call(
        body,
        grid=(nn // _BN,),
        in_specs=[
            pl.BlockSpec((_NC, _BN, d), lambda i: (0, i, 0)),
            pl.BlockSpec((lat, k), lambda i: (0, 0)),
            pl.BlockSpec((lat, k), lambda i: (0, 0)),
            pl.BlockSpec((1, k), lambda i: (0, 0)),
        ],
        out_specs=[
            pl.BlockSpec((_BN, k), lambda i: (i, 0)),
            pl.BlockSpec((_BN, k), lambda i: (i, 0)),
            pl.BlockSpec((_BN, 1), lambda i: (i, 0)),
        ],
        out_shape=[
            jax.ShapeDtypeStruct((nn, k), jnp.float32),
            jax.ShapeDtypeStruct((nn, k), jnp.float32),
            jax.ShapeDtypeStruct((nn, 1), jnp.float32),
        ],
    )(p, ea, eb, db0)


def _final_tc(pd, cnt, w3, b3):
    nn = pd.shape[1]
    k = pd.shape[2]
    dout = w3.shape[1]

    def body(pd_ref, cnt_ref, w3_ref, b3_ref, out_ref):
        s = pd_ref[0] + pd_ref[1]
        c = cnt_ref[...]
        o = _dot(s / jnp.maximum(c, 1.0), w3_ref[...]) + b3_ref[...]
        out_ref[...] = jnp.where(c > 0.0, o, 0.0)

    return pl.pallas_call(
        body,
        grid=(nn // _BN,),
        in_specs=[
            pl.BlockSpec((_NC, _BN, k), lambda i: (0, i, 0)),
            pl.BlockSpec((_BN, 1), lambda i: (i, 0)),
            pl.BlockSpec((k, dout), lambda i: (0, 0)),
            pl.BlockSpec((1, dout), lambda i: (0, 0)),
        ],
        out_specs=pl.BlockSpec((_BN, dout), lambda i: (i, 0)),
        out_shape=jax.ShapeDtypeStruct((nn, dout), jnp.float32),
    )(pd, cnt, w3, b3)


def _sc_gather_pair(taba, tabb, dstg, srcg):
    nn, k = taba.shape
    ne = dstg.shape[0] * dstg.shape[1]
    ech = ne // _NW
    nch = ech // _C
    ng = nch // _G
    kp = _PK * k
    rch = _C // _PK
    mesh = plsc.VectorSubcoreMesh(core_axis_name="c", subcore_axis_name="s")

    @functools.partial(
        pl.kernel,
        out_type=(jax.ShapeDtypeStruct((ne, k), jnp.float32),
                  jax.ShapeDtypeStruct((ne, k), jnp.float32)),
        mesh=mesh,
        compiler_params=pltpu.CompilerParams(use_tc_tiling_on_sc=False),
        scratch_types=[
            pltpu.VMEM((nch, _C), jnp.int32),
            pltpu.VMEM((nch, _C), jnp.int32),
            pltpu.VMEM((_G, _C, k), jnp.float32),
            pltpu.VMEM((_G, _C, k), jnp.float32),
            pltpu.SemaphoreType.DMA,
            pltpu.SemaphoreType.DMA,
            pltpu.SemaphoreType.DMA,
        ],
    )
    def run(taba_h, tabb_h, dst_h, src_h, outa_h, outb_h,
            idxd, idxs, bufa, bufb, sema, semb, semw):
        wid = lax.axis_index("s") * _NC + lax.axis_index("c")
        cbase = wid * nch
        ebase = wid * ech
        pltpu.sync_copy(dst_h.at[pl.ds(cbase, nch)], idxd)
        pltpu.sync_copy(src_h.at[pl.ds(cbase, nch)], idxs)

        def group(g, carry):
            ha = []
            hb = []
            for b in range(_G):
                j = g * _G + b
                ha.append(pltpu.async_copy(taba_h.at[idxd.at[j]], bufa.at[b],
                                           sema))
                hb.append(pltpu.async_copy(tabb_h.at[idxs.at[j]], bufb.at[b],
                                           semb))
            pend = []
            for b in range(_G):
                j = g * _G + b
                ha[b].wait()
                pend.append(pltpu.async_copy(
                    bufa.at[b],
                    outa_h.at[pl.ds(ebase + j * _C, _C)], semw))
                hb[b].wait()
                pend.append(pltpu.async_copy(
                    bufb.at[b],
                    outb_h.at[pl.ds(ebase + j * _C, _C)], semw))
            for w in pend:
                w.wait()
            return carry

        lax.fori_loop(0, ng, group, 0)

    return run(taba, tabb, dstg, srcg)


def _sc_scatter_add(h, dstg, nn):
    ne, d = h.shape
    ech = ne // _NW
    nch = ech // _C
    rpt = nn // _NS
    mesh = plsc.VectorSubcoreMesh(core_axis_name="c", subcore_axis_name="s")

    @functools.partial(
        pl.kernel,
        out_type=jax.ShapeDtypeStruct((_NC, nn, d), jnp.float32),
        mesh=mesh,
        compiler_params=pltpu.CompilerParams(use_tc_tiling_on_sc=False),
        scratch_types=[
            pltpu.VMEM((nch, _C), jnp.int32),
            pltpu.VMEM((2, _C, d), jnp.float32),
            pltpu.VMEM((rpt, d), jnp.float32),
            pltpu.VMEM_SHARED((nn, d), jnp.float32),
            pltpu.SemaphoreType.DMA,
        ],
    )
    def run(h_h, dst_h, out_h, idxd, hbuf, zbuf, acc, seml):
        cid = lax.axis_index("c")
        sid = lax.axis_index("s")
        wid = sid * _NC + cid
        cbase = wid * nch
        ebase = wid * ech

        zv = jnp.zeros((16,), jnp.float32)

        def zrow(i, carry):
            for t in range(d // 16):
                zbuf[i, pl.ds(t * 16, 16)] = zv
            return carry

        lax.fori_loop(0, rpt, zrow, 0)
        pltpu.sync_copy(zbuf, acc.at[pl.ds(sid * rpt, rpt)])
        pltpu.sync_copy(dst_h.at[pl.ds(cbase, nch)], idxd)
        plsc.subcore_barrier()

        def lstart(j, b):
            return pltpu.async_copy(h_h.at[pl.ds(ebase + j * _C, _C)],
                                    hbuf.at[b], seml)

        def lwait(j, b):
            pltpu.make_async_copy(h_h.at[pl.ds(ebase + j * _C, _C)],
                                  hbuf.at[b], seml).wait()

        def sadd(j, b):
            pltpu.sync_copy(hbuf.at[b], acc.at[idxd.at[j]], add=True)

        lstart(0, 0)

        def pair(g, carry):
            j0 = 2 * g
            lstart(j0 + 1, 1)
            lwait(j0, 0)
            sadd(j0, 0)
            lstart(j0 + 2, 0)
            lwait(j0 + 1, 1)
            sadd(j0 + 1, 1)
            return carry

        lax.fori_loop(0, (nch - 1) // 2, pair, 0)
        lwait(nch - 1, 0)
        sadd(nch - 1, 0)

        plsc.subcore_barrier()
        pltpu.sync_copy(acc.at[pl.ds(sid * rpt, rpt)],
                        out_h.at[cid, pl.ds(sid * rpt, rpt)])

    return run(h, dstg)


def kernel(x, edge_index,
           enc_W0, enc_b0, enc_W1, enc_b1, enc_W2, enc_b2, enc_W3, enc_b3,
           dec_W0, dec_b0, dec_W1, dec_b1, dec_W2, dec_b2, dec_W3, dec_b3):
    nn, din = x.shape
    ne = edge_index.shape[1]
    lat = enc_W3.shape[1]

    src = edge_index[0]
    dst = edge_index[1]
    dstg = dst.reshape(ne // _C, _C)
    srcg = src.reshape(ne // _C, _C)

    f32 = jnp.float32
    wa = enc_W0[:din] - enc_W0[din:]
    wb = enc_W0[din:]
    b0 = enc_b0.reshape(1, -1)
    b1 = enc_b1.reshape(1, -1)
    b2 = enc_b2.reshape(1, -1)
    w3p = jnp.concatenate([enc_W3, jnp.zeros((enc_W3.shape[0], 16 - lat), f32)],
                          axis=1)
    b3p = jnp.concatenate(
        [enc_b3, jnp.ones((1,), f32), jnp.zeros((16 - lat - 1,), f32)]
    ).reshape(1, 16)

    ea = dec_W0[:lat] - dec_W0[lat:]
    eb = dec_W0[lat:]
    db0 = dec_b0.reshape(1, -1)
    db1 = dec_b1.reshape(1, -1)
    db2 = dec_b2.reshape(1, -1)
    db3 = dec_b3.reshape(1, -1)

    w1b, w2b, w3pb = _blkdiag(enc_W1), _blkdiag(enc_W2), _blkdiag(w3p)
    b1b, b2b, b3pb = (jnp.tile(b1, (1, _PK)), jnp.tile(b2, (1, _PK)),
                      jnp.tile(b3p, (1, _PK)))
    dw1b, dw2b = _blkdiag(dec_W1), _blkdiag(dec_W2)
    db1b, db2b = jnp.tile(db1, (1, _PK)), jnp.tile(db2, (1, _PK))

    u, v = _node_tables_tc(x, wa, wb, b0)
    gu, gv = _sc_gather_pair(u, v, dstg, srcg)
    gu8 = gu.reshape(ne // _PK, _PK * 32)
    gv8 = gv.reshape(ne // _PK, _PK * 32)
    e4 = _edge_mlp_x3_tc(
        gu8, gv8,
        [_split_bf16(w1b), _split_bf16(w2b), _split_bf16(w3pb)],
        [b1b, b2b, b3pb])
    p = _sc_scatter_add(e4.reshape(ne, 16), dstg, nn)
    ud, vd, cnt = _dec_tables_tc(p, ea, eb, db0)
    gud, gvd = _sc_gather_pair(ud, vd, dstg, srcg)
    gud8 = gud.reshape(ne // _PK, _PK * 32)
    gvd8 = gvd.reshape(ne // _PK, _PK * 32)
    h3 = _edge_mlp_tc(gud8, gvd8, dw1b, db1b, dw2b, db2b,
                      prec=lax.Precision.DEFAULT)
    pd = _sc_scatter_add(h3.reshape(ne, 32), dstg, nn)
    return _final_tc(pd, cnt, dec_W3, db3)


# speedup vs baseline: 8.5531x; 1.1265x over previous
import functools

import jax
import jax.numpy as jnp
from jax import lax
from jax.experimental import pallas as pl
from jax.experimental.pallas import tpu as pltpu
from jax.experimental.pallas import tpu_sc as plsc

_NC = 2
_NS = 16
_NW = _NC * _NS
_C = 80
_G = 5

_BN = 2000
_BE = 8000


def _dot(a, b, prec=lax.Precision.HIGHEST):
    return lax.dot_general(a, b, (((1,), (0,)), ((), ())),
                           precision=prec,
                           preferred_element_type=jnp.float32)


def _node_tables_tc(x, wa, wb, b0):
    nn, d = x.shape
    k = wa.shape[1]

    def body(x_ref, wa_ref, wb_ref, b0_ref, u_ref, v_ref):
        xb = x_ref[...]
        u_ref[...] = _dot(xb, wa_ref[...]) + b0_ref[...]
        v_ref[...] = _dot(xb, wb_ref[...])

    return pl.pallas_call(
        body,
        grid=(nn // _BN,),
        in_specs=[
            pl.BlockSpec((_BN, d), lambda i: (i, 0)),
            pl.BlockSpec((d, k), lambda i: (0, 0)),
            pl.BlockSpec((d, k), lambda i: (0, 0)),
            pl.BlockSpec((1, k), lambda i: (0, 0)),
        ],
        out_specs=[
            pl.BlockSpec((_BN, k), lambda i: (i, 0)),
            pl.BlockSpec((_BN, k), lambda i: (i, 0)),
        ],
        out_shape=[
            jax.ShapeDtypeStruct((nn, k), jnp.float32),
            jax.ShapeDtypeStruct((nn, k), jnp.float32),
        ],
    )(x, wa, wb, b0)


_PK = 8


def _blkdiag(w):
    fi, fo = w.shape
    out = jnp.zeros((_PK * fi, _PK * fo), jnp.float32)
    for t in range(_PK):
        out = lax.dynamic_update_slice(out, w, (t * fi, t * fo))
    return out


def _split_bf16(w):
    hi = w.astype(jnp.bfloat16)
    lo = (w - hi.astype(jnp.float32)).astype(jnp.bfloat16)
    return hi, lo


def _edge_mlp_x3_tc(gu, gv, ws, bs):
    nr, k = gu.shape
    br = _BE // _PK
    nl = len(ws)

    def body(*refs):
        gu_ref, gv_ref = refs[0], refs[1]
        out_ref = refs[-1]
        h = jnp.maximum(gu_ref[...] + gv_ref[...], 0.0)
        for i in range(nl):
            whi = refs[2 + 3 * i][...]
            wlo = refs[3 + 3 * i][...]
            b = refs[4 + 3 * i][...]
            hhi = h.astype(jnp.bfloat16)
            hlo = (h - hhi.astype(jnp.float32)).astype(jnp.bfloat16)
            acc = _dot(hhi, whi, lax.Precision.DEFAULT)
            acc = acc + _dot(hlo, whi, lax.Precision.DEFAULT)
            acc = acc + _dot(hhi, wlo, lax.Precision.DEFAULT)
            h = jnp.maximum(acc + b, 0.0)
        out_ref[...] = h

    in_specs = [
        pl.BlockSpec((br, k), lambda i: (i, 0)),
        pl.BlockSpec((br, k), lambda i: (i, 0)),
    ]
    args = [gu, gv]
    for (whi, wlo), b in zip(ws, bs):
        ki, ko = whi.shape
        in_specs += [pl.BlockSpec((ki, ko), lambda i: (0, 0)),
                     pl.BlockSpec((ki, ko), lambda i: (0, 0)),
                     pl.BlockSpec((1, ko), lambda i: (0, 0))]
        args += [whi, wlo, b]
    kout = ws[-1][0].shape[1]
    return pl.pallas_call(
        body,
        grid=(nr // br,),
        in_specs=in_specs,
        out_specs=pl.BlockSpec((br, kout), lambda i: (i, 0)),
        out_shape=jax.ShapeDtypeStruct((nr, kout), jnp.float32),
    )(*args)


def _edge_mlp_tc(gu, gv, w1, b1, w2, b2, w3=None, b3=None,
                 prec=lax.Precision.HIGHEST, first=None):
    nr, k = gu.shape
    k1 = w1.shape[1]
    k2 = w2.shape[1]
    kout = w3.shape[1] if w3 is not None else k2
    three = w3 is not None
    br = _BE // _PK
    nf = 3 if first is not None else 0

    def body(gu_ref, gv_ref, *rest):
        out_ref = rest[-1]
        if first is not None:
            h = jnp.maximum(_dot(gu_ref[...], rest[0][...], prec)
                            + _dot(gv_ref[...], rest[1][...], prec)
                            + rest[2][...], 0.0)
        else:
            h = jnp.maximum(gu_ref[...] + gv_ref[...], 0.0)
        h = jnp.maximum(_dot(h, rest[nf][...], prec) + rest[nf + 1][...], 0.0)
        h = jnp.maximum(_dot(h, rest[nf + 2][...], prec) + rest[nf + 3][...],
                        0.0)
        if three:
            h = jnp.maximum(_dot(h, rest[nf + 4][...], prec)
                            + rest[nf + 5][...], 0.0)
        out_ref[...] = h

    in_specs = [
        pl.BlockSpec((br, k), lambda i: (i, 0)),
        pl.BlockSpec((br, k), lambda i: (i, 0)),
    ]
    args = [gu, gv]
    if first is not None:
        wa, wb, b0f = first
        k0 = wa.shape[1]
        in_specs += [pl.BlockSpec((k, k0), lambda i: (0, 0)),
                     pl.BlockSpec((k, k0), lambda i: (0, 0)),
                     pl.BlockSpec((1, k0), lambda i: (0, 0))]
        args += [wa, wb, b0f]
    in_specs += [
        pl.BlockSpec((w1.shape[0], k1), lambda i: (0, 0)),
        pl.BlockSpec((1, k1), lambda i: (0, 0)),
        pl.BlockSpec((k1, k2), lambda i: (0, 0)),
        pl.BlockSpec((1, k2), lambda i: (0, 0)),
    ]
    args += [w1, b1, w2, b2]
    if three:
        in_specs += [pl.BlockSpec((k2, kout), lambda i: (0, 0)),
                     pl.BlockSpec((1, kout), lambda i: (0, 0))]
        args += [w3, b3]
    return pl.pallas_call(
        body,
        grid=(nr // br,),
        in_specs=in_specs,
        out_specs=pl.BlockSpec((br, kout), lambda i: (i, 0)),
        out_shape=jax.ShapeDtypeStruct((nr, kout), jnp.float32),
    )(*args)


def _dec_tables_tc(p, lat):
    nn = p.shape[1]
    d = p.shape[2]

    def body(p_ref, z_ref, cnt_ref):
        s = p_ref[0] + p_ref[1]
        cnt = s[:, lat:lat + 1]
        z = s[:, 0:lat] / jnp.maximum(cnt, 1.0)
        z_ref[...] = jnp.pad(z, ((0, 0), (0, 8 - lat)))
        cnt_ref[...] = cnt

    return pl.pallas_call(
        body,
        grid=(nn // _BN,),
        in_specs=[
            pl.BlockSpec((_NC, _BN, d), lambda i: (0, i, 0)),
        ],
        out_specs=[
            pl.BlockSpec((_BN, 8), lambda i: (i, 0)),
            pl.BlockSpec((_BN, 1), lambda i: (i, 0)),
        ],
        out_shape=[
            jax.ShapeDtypeStruct((nn, 8), jnp.float32),
            jax.ShapeDtypeStruct((nn, 1), jnp.float32),
        ],
    )(p)


def _final_tc(pd, cnt, w3, b3):
    nn = pd.shape[1]
    k = pd.shape[2]
    dout = w3.shape[1]

    def body(pd_ref, cnt_ref, w3_ref, b3_ref, out_ref):
        s = pd_ref[0] + pd_ref[1]
        c = cnt_ref[...]
        o = _dot(s / jnp.maximum(c, 1.0), w3_ref[...]) + b3_ref[...]
        out_ref[...] = jnp.where(c > 0.0, o, 0.0)

    return pl.pallas_call(
        body,
        grid=(nn // _BN,),
        in_specs=[
            pl.BlockSpec((_NC, _BN, k), lambda i: (0, i, 0)),
            pl.BlockSpec((_BN, 1), lambda i: (i, 0)),
            pl.BlockSpec((k, dout), lambda i: (0, 0)),
            pl.BlockSpec((1, dout), lambda i: (0, 0)),
        ],
        out_specs=pl.BlockSpec((_BN, dout), lambda i: (i, 0)),
        out_shape=jax.ShapeDtypeStruct((nn, dout), jnp.float32),
    )(pd, cnt, w3, b3)


def _sc_gather_pair(taba, tabb, dstg, srcg):
    nn, k = taba.shape
    ne = dstg.shape[0] * dstg.shape[1]
    ech = ne // _NW
    nch = ech // _C
    ng = nch // _G
    mesh = plsc.VectorSubcoreMesh(core_axis_name="c", subcore_axis_name="s")

    @functools.partial(
        pl.kernel,
        out_type=(jax.ShapeDtypeStruct((ne, k), jnp.float32),
                  jax.ShapeDtypeStruct((ne, k), jnp.float32)),
        mesh=mesh,
        compiler_params=pltpu.CompilerParams(use_tc_tiling_on_sc=False),
        scratch_types=[
            pltpu.VMEM((nch, _C), jnp.int32),
            pltpu.VMEM((nch, _C), jnp.int32),
            pltpu.VMEM((_G, _C, k), jnp.float32),
            pltpu.VMEM((_G, _C, k), jnp.float32),
            pltpu.SemaphoreType.DMA,
            pltpu.SemaphoreType.DMA,
            pltpu.SemaphoreType.DMA,
        ],
    )
    def run(taba_h, tabb_h, dst_h, src_h, outa_h, outb_h,
            idxd, idxs, bufa, bufb, sema, semb, semw):
        wid = lax.axis_index("s") * _NC + lax.axis_index("c")
        cbase = wid * nch
        ebase = wid * ech
        pltpu.sync_copy(dst_h.at[pl.ds(cbase, nch)], idxd)
        pltpu.sync_copy(src_h.at[pl.ds(cbase, nch)], idxs)

        def group(g, carry):
            ha = []
            hb = []
            for b in range(_G):
                j = g * _G + b
                ha.append(pltpu.async_copy(taba_h.at[idxd.at[j]], bufa.at[b],
                                           sema))
                hb.append(pltpu.async_copy(tabb_h.at[idxs.at[j]], bufb.at[b],
                                           semb))
            pend = []
            for b in range(_G):
                j = g * _G + b
                ha[b].wait()
                pend.append(pltpu.async_copy(
                    bufa.at[b],
                    outa_h.at[pl.ds(ebase + j * _C, _C)], semw))
                hb[b].wait()
                pend.append(pltpu.async_copy(
                    bufb.at[b],
                    outb_h.at[pl.ds(ebase + j * _C, _C)], semw))
            for w in pend:
                w.wait()
            return carry

        lax.fori_loop(0, ng, group, 0)

    return run(taba, tabb, dstg, srcg)


def _sc_scatter_add(h, dstg, nn):
    ne, d = h.shape
    ech = ne // _NW
    nch = ech // _C
    rpt = nn // _NS
    mesh = plsc.VectorSubcoreMesh(core_axis_name="c", subcore_axis_name="s")

    @functools.partial(
        pl.kernel,
        out_type=jax.ShapeDtypeStruct((_NC, nn, d), jnp.float32),
        mesh=mesh,
        compiler_params=pltpu.CompilerParams(use_tc_tiling_on_sc=False),
        scratch_types=[
            pltpu.VMEM((nch, _C), jnp.int32),
            pltpu.VMEM((2, _C, d), jnp.float32),
            pltpu.VMEM((rpt, d), jnp.float32),
            pltpu.VMEM_SHARED((nn, d), jnp.float32),
            pltpu.SemaphoreType.DMA,
        ],
    )
    def run(h_h, dst_h, out_h, idxd, hbuf, zbuf, acc, seml):
        cid = lax.axis_index("c")
        sid = lax.axis_index("s")
        wid = sid * _NC + cid
        cbase = wid * nch
        ebase = wid * ech

        zv = jnp.zeros((16,), jnp.float32)

        def zrow(i, carry):
            for t in range(d // 16):
                zbuf[i, pl.ds(t * 16, 16)] = zv
            return carry

        lax.fori_loop(0, rpt, zrow, 0)
        pltpu.sync_copy(zbuf, acc.at[pl.ds(sid * rpt, rpt)])
        pltpu.sync_copy(dst_h.at[pl.ds(cbase, nch)], idxd)
        plsc.subcore_barrier()

        def lstart(j, b):
            return pltpu.async_copy(h_h.at[pl.ds(ebase + j * _C, _C)],
                                    hbuf.at[b], seml)

        def lwait(j, b):
            pltpu.make_async_copy(h_h.at[pl.ds(ebase + j * _C, _C)],
                                  hbuf.at[b], seml).wait()

        def sadd(j, b):
            pltpu.sync_copy(hbuf.at[b], acc.at[idxd.at[j]], add=True)

        lstart(0, 0)

        def pair(g, carry):
            j0 = 2 * g
            lstart(j0 + 1, 1)
            lwait(j0, 0)
            sadd(j0, 0)
            lstart(j0 + 2, 0)
            lwait(j0 + 1, 1)
            sadd(j0 + 1, 1)
            return carry

        lax.fori_loop(0, (nch - 1) // 2, pair, 0)
        lwait(nch - 1, 0)
        sadd(nch - 1, 0)

        plsc.subcore_barrier()
        pltpu.sync_copy(acc.at[pl.ds(sid * rpt, rpt)],
                        out_h.at[cid, pl.ds(sid * rpt, rpt)])

    return run(h, dstg)


def kernel(x, edge_index,
           enc_W0, enc_b0, enc_W1, enc_b1, enc_W2, enc_b2, enc_W3, enc_b3,
           dec_W0, dec_b0, dec_W1, dec_b1, dec_W2, dec_b2, dec_W3, dec_b3):
    nn, din = x.shape
    ne = edge_index.shape[1]
    lat = enc_W3.shape[1]

    src = edge_index[0]
    dst = edge_index[1]
    dstg = dst.reshape(ne // _C, _C)
    srcg = src.reshape(ne // _C, _C)

    f32 = jnp.float32
    wa = enc_W0[:din] - enc_W0[din:]
    wb = enc_W0[din:]
    b0 = enc_b0.reshape(1, -1)
    b1 = enc_b1.reshape(1, -1)
    b2 = enc_b2.reshape(1, -1)
    w3p = jnp.concatenate([enc_W3, jnp.zeros((enc_W3.shape[0], 16 - lat), f32)],
                          axis=1)
    b3p = jnp.concatenate(
        [enc_b3, jnp.ones((1,), f32), jnp.zeros((16 - lat - 1,), f32)]
    ).reshape(1, 16)

    eap = jnp.pad(dec_W0[:lat] - dec_W0[lat:], ((0, 8 - lat), (0, 0)))
    ebp = jnp.pad(dec_W0[lat:], ((0, 8 - lat), (0, 0)))
    db0 = dec_b0.reshape(1, -1)
    db1 = dec_b1.reshape(1, -1)
    db2 = dec_b2.reshape(1, -1)
    db3 = dec_b3.reshape(1, -1)

    w1b, w2b, w3pb = _blkdiag(enc_W1), _blkdiag(enc_W2), _blkdiag(w3p)
    b1b, b2b, b3pb = (jnp.tile(b1, (1, _PK)), jnp.tile(b2, (1, _PK)),
                      jnp.tile(b3p, (1, _PK)))
    dw1b, dw2b = _blkdiag(dec_W1), _blkdiag(dec_W2)
    db1b, db2b = jnp.tile(db1, (1, _PK)), jnp.tile(db2, (1, _PK))
    eab, ebb = _blkdiag(eap), _blkdiag(ebp)
    db0b = jnp.tile(db0, (1, _PK))

    u, v = _node_tables_tc(x, wa, wb, b0)
    gu, gv = _sc_gather_pair(u, v, dstg, srcg)
    gu8 = gu.reshape(ne // _PK, _PK * 32)
    gv8 = gv.reshape(ne // _PK, _PK * 32)
    e4 = _edge_mlp_x3_tc(
        gu8, gv8,
        [_split_bf16(w1b), _split_bf16(w2b), _split_bf16(w3pb)],
        [b1b, b2b, b3pb])
    p = _sc_scatter_add(e4.reshape(ne, 16), dstg, nn)
    z8, cnt = _dec_tables_tc(p, lat)
    zd, zs = _sc_gather_pair(z8, z8, dstg, srcg)
    zd8 = zd.reshape(ne // _PK, _PK * 8)
    zs8 = zs.reshape(ne // _PK, _PK * 8)
    h3 = _edge_mlp_tc(zd8, zs8, dw1b, db1b, dw2b, db2b,
                      prec=lax.Precision.DEFAULT,
                      first=(eab, ebb, db0b))
    pd = _sc_scatter_add(h3.reshape(ne, 32), dstg, nn)
    return _final_tc(pd, cnt, dec_W3, db3)


# speedup vs baseline: 9.4081x; 1.1000x over previous
import functools

import jax
import jax.numpy as jnp
from jax import lax
from jax.experimental import pallas as pl
from jax.experimental.pallas import tpu as pltpu
from jax.experimental.pallas import tpu_sc as plsc

_NC = 2
_NS = 16
_NW = _NC * _NS
_C = 80
_G = 5

_BN = 2000
_BE = 16000


def _dot(a, b, prec=lax.Precision.HIGHEST):
    return lax.dot_general(a, b, (((1,), (0,)), ((), ())),
                           precision=prec,
                           preferred_element_type=jnp.float32)


def _node_tables_tc(x, wa, wb, b0):
    nn, d = x.shape
    k = wa.shape[1]

    def body(x_ref, wa_ref, wb_ref, b0_ref, u_ref, v_ref):
        xb = x_ref[...]
        u_ref[...] = _dot(xb, wa_ref[...]) + b0_ref[...]
        v_ref[...] = _dot(xb, wb_ref[...])

    return pl.pallas_call(
        body,
        grid=(nn // _BN,),
        in_specs=[
            pl.BlockSpec((_BN, d), lambda i: (i, 0)),
            pl.BlockSpec((d, k), lambda i: (0, 0)),
            pl.BlockSpec((d, k), lambda i: (0, 0)),
            pl.BlockSpec((1, k), lambda i: (0, 0)),
        ],
        out_specs=[
            pl.BlockSpec((_BN, k), lambda i: (i, 0)),
            pl.BlockSpec((_BN, k), lambda i: (i, 0)),
        ],
        out_shape=[
            jax.ShapeDtypeStruct((nn, k), jnp.float32),
            jax.ShapeDtypeStruct((nn, k), jnp.float32),
        ],
    )(x, wa, wb, b0)


_PK = 8


def _blkdiag(w):
    fi, fo = w.shape
    out = jnp.zeros((_PK * fi, _PK * fo), jnp.float32)
    for t in range(_PK):
        out = lax.dynamic_update_slice(out, w, (t * fi, t * fo))
    return out


def _split_bf16(w):
    hi = w.astype(jnp.bfloat16)
    lo = (w - hi.astype(jnp.float32)).astype(jnp.bfloat16)
    return hi, lo


def _edge_mlp_x3_tc(gu, gv, ws, bs):
    nr, k = gu.shape
    br = _BE // _PK
    nl = len(ws)

    def body(*refs):
        gu_ref, gv_ref = refs[0], refs[1]
        out_ref = refs[-1]
        h = jnp.maximum(gu_ref[...] + gv_ref[...], 0.0)
        for i in range(nl):
            whi = refs[2 + 3 * i][...]
            wlo = refs[3 + 3 * i][...]
            b = refs[4 + 3 * i][...]
            hhi = h.astype(jnp.bfloat16)
            hlo = (h - hhi.astype(jnp.float32)).astype(jnp.bfloat16)
            acc = _dot(hhi, whi, lax.Precision.DEFAULT)
            acc = acc + _dot(hlo, whi, lax.Precision.DEFAULT)
            acc = acc + _dot(hhi, wlo, lax.Precision.DEFAULT)
            h = jnp.maximum(acc + b, 0.0)
        out_ref[...] = h

    in_specs = [
        pl.BlockSpec((br, k), lambda i: (i, 0)),
        pl.BlockSpec((br, k), lambda i: (i, 0)),
    ]
    args = [gu, gv]
    for (whi, wlo), b in zip(ws, bs):
        ki, ko = whi.shape
        in_specs += [pl.BlockSpec((ki, ko), lambda i: (0, 0)),
                     pl.BlockSpec((ki, ko), lambda i: (0, 0)),
                     pl.BlockSpec((1, ko), lambda i: (0, 0))]
        args += [whi, wlo, b]
    kout = ws[-1][0].shape[1]
    return pl.pallas_call(
        body,
        grid=(nr // br,),
        in_specs=in_specs,
        out_specs=pl.BlockSpec((br, kout), lambda i: (i, 0)),
        out_shape=jax.ShapeDtypeStruct((nr, kout), jnp.float32),
    )(*args)


def _edge_mlp_tc(gu, gv, w1, b1, w2, b2, w3=None, b3=None,
                 prec=lax.Precision.HIGHEST, first=None):
    nr, k = gu.shape
    k1 = w1.shape[1]
    k2 = w2.shape[1]
    kout = w3.shape[1] if w3 is not None else k2
    three = w3 is not None
    br = _BE // _PK
    nf = 3 if first is not None else 0

    def body(gu_ref, gv_ref, *rest):
        out_ref = rest[-1]
        if first is not None:
            h = jnp.maximum(_dot(gu_ref[...], rest[0][...], prec)
                            + _dot(gv_ref[...], rest[1][...], prec)
                            + rest[2][...], 0.0)
        else:
            h = jnp.maximum(gu_ref[...] + gv_ref[...], 0.0)
        h = jnp.maximum(_dot(h, rest[nf][...], prec) + rest[nf + 1][...], 0.0)
        h = jnp.maximum(_dot(h, rest[nf + 2][...], prec) + rest[nf + 3][...],
                        0.0)
        if three:
            h = jnp.maximum(_dot(h, rest[nf + 4][...], prec)
                            + rest[nf + 5][...], 0.0)
        out_ref[...] = h

    in_specs = [
        pl.BlockSpec((br, k), lambda i: (i, 0)),
        pl.BlockSpec((br, k), lambda i: (i, 0)),
    ]
    args = [gu, gv]
    if first is not None:
        wa, wb, b0f = first
        k0 = wa.shape[1]
        in_specs += [pl.BlockSpec((k, k0), lambda i: (0, 0)),
                     pl.BlockSpec((k, k0), lambda i: (0, 0)),
                     pl.BlockSpec((1, k0), lambda i: (0, 0))]
        args += [wa, wb, b0f]
    in_specs += [
        pl.BlockSpec((w1.shape[0], k1), lambda i: (0, 0)),
        pl.BlockSpec((1, k1), lambda i: (0, 0)),
        pl.BlockSpec((k1, k2), lambda i: (0, 0)),
        pl.BlockSpec((1, k2), lambda i: (0, 0)),
    ]
    args += [w1, b1, w2, b2]
    if three:
        in_specs += [pl.BlockSpec((k2, kout), lambda i: (0, 0)),
                     pl.BlockSpec((1, kout), lambda i: (0, 0))]
        args += [w3, b3]
    return pl.pallas_call(
        body,
        grid=(nr // br,),
        in_specs=in_specs,
        out_specs=pl.BlockSpec((br, kout), lambda i: (i, 0)),
        out_shape=jax.ShapeDtypeStruct((nr, kout), jnp.float32),
    )(*args)


def _dec_tables_tc(p, lat):
    nn = p.shape[1]
    d = p.shape[2]

    def body(p_ref, z_ref, cnt_ref):
        s = p_ref[0] + p_ref[1]
        cnt = s[:, lat:lat + 1]
        z = s[:, 0:lat] / jnp.maximum(cnt, 1.0)
        z_ref[...] = jnp.pad(z, ((0, 0), (0, 8 - lat)))
        cnt_ref[...] = cnt

    return pl.pallas_call(
        body,
        grid=(nn // _BN,),
        in_specs=[
            pl.BlockSpec((_NC, _BN, d), lambda i: (0, i, 0)),
        ],
        out_specs=[
            pl.BlockSpec((_BN, 8), lambda i: (i, 0)),
            pl.BlockSpec((_BN, 1), lambda i: (i, 0)),
        ],
        out_shape=[
            jax.ShapeDtypeStruct((nn, 8), jnp.float32),
            jax.ShapeDtypeStruct((nn, 1), jnp.float32),
        ],
    )(p)


def _final_tc(pd, cnt, w3, b3):
    nn = pd.shape[1]
    k = pd.shape[2]
    dout = w3.shape[1]

    def body(pd_ref, cnt_ref, w3_ref, b3_ref, out_ref):
        s = pd_ref[0] + pd_ref[1]
        c = cnt_ref[...]
        o = _dot(s / jnp.maximum(c, 1.0), w3_ref[...]) + b3_ref[...]
        out_ref[...] = jnp.where(c > 0.0, o, 0.0)

    return pl.pallas_call(
        body,
        grid=(nn // _BN,),
        in_specs=[
            pl.BlockSpec((_NC, _BN, k), lambda i: (0, i, 0)),
            pl.BlockSpec((_BN, 1), lambda i: (i, 0)),
            pl.BlockSpec((k, dout), lambda i: (0, 0)),
            pl.BlockSpec((1, dout), lambda i: (0, 0)),
        ],
        out_specs=pl.BlockSpec((_BN, dout), lambda i: (i, 0)),
        out_shape=jax.ShapeDtypeStruct((nn, dout), jnp.float32),
    )(pd, cnt, w3, b3)


def _sc_gather_pair(taba, tabb, dstg, srcg):
    nn, k = taba.shape
    ne = dstg.shape[0] * dstg.shape[1]
    ech = ne // _NW
    nch = ech // _C
    ng = nch // _G
    mesh = plsc.VectorSubcoreMesh(core_axis_name="c", subcore_axis_name="s")

    @functools.partial(
        pl.kernel,
        out_type=(jax.ShapeDtypeStruct((ne, k), jnp.float32),
                  jax.ShapeDtypeStruct((ne, k), jnp.float32)),
        mesh=mesh,
        compiler_params=pltpu.CompilerParams(use_tc_tiling_on_sc=False),
        scratch_types=[
            pltpu.VMEM((nch, _C), jnp.int32),
            pltpu.VMEM((nch, _C), jnp.int32),
            pltpu.VMEM((_G, _C, k), jnp.float32),
            pltpu.VMEM((_G, _C, k), jnp.float32),
            pltpu.SemaphoreType.DMA,
            pltpu.SemaphoreType.DMA,
            pltpu.SemaphoreType.DMA,
        ],
    )
    def run(taba_h, tabb_h, dst_h, src_h, outa_h, outb_h,
            idxd, idxs, bufa, bufb, sema, semb, semw):
        wid = lax.axis_index("s") * _NC + lax.axis_index("c")
        cbase = wid * nch
        ebase = wid * ech
        pltpu.sync_copy(dst_h.at[pl.ds(cbase, nch)], idxd)
        pltpu.sync_copy(src_h.at[pl.ds(cbase, nch)], idxs)

        def group(g, carry):
            ha = []
            hb = []
            for b in range(_G):
                j = g * _G + b
                ha.append(pltpu.async_copy(taba_h.at[idxd.at[j]], bufa.at[b],
                                           sema))
                hb.append(pltpu.async_copy(tabb_h.at[idxs.at[j]], bufb.at[b],
                                           semb))
            pend = []
            for b in range(_G):
                j = g * _G + b
                ha[b].wait()
                pend.append(pltpu.async_copy(
                    bufa.at[b],
                    outa_h.at[pl.ds(ebase + j * _C, _C)], semw))
                hb[b].wait()
                pend.append(pltpu.async_copy(
                    bufb.at[b],
                    outb_h.at[pl.ds(ebase + j * _C, _C)], semw))
            for w in pend:
                w.wait()
            return carry

        lax.fori_loop(0, ng, group, 0)

    return run(taba, tabb, dstg, srcg)


def _sc_scatter_add(h, dstg, nn):
    ne, d = h.shape
    ech = ne // _NW
    nch = ech // _C
    rpt = nn // _NS
    mesh = plsc.VectorSubcoreMesh(core_axis_name="c", subcore_axis_name="s")

    @functools.partial(
        pl.kernel,
        out_type=jax.ShapeDtypeStruct((_NC, nn, d), jnp.float32),
        mesh=mesh,
        compiler_params=pltpu.CompilerParams(use_tc_tiling_on_sc=False),
        scratch_types=[
            pltpu.VMEM((nch, _C), jnp.int32),
            pltpu.VMEM((4, _C, d), jnp.float32),
            pltpu.VMEM((rpt, d), jnp.float32),
            pltpu.VMEM_SHARED((nn, d), jnp.float32),
            pltpu.SemaphoreType.DMA,
            pltpu.SemaphoreType.DMA,
        ],
    )
    def run(h_h, dst_h, out_h, idxd, hbuf, zbuf, acc, seml, sema):
        cid = lax.axis_index("c")
        sid = lax.axis_index("s")
        wid = sid * _NC + cid
        cbase = wid * nch
        ebase = wid * ech

        zv = jnp.zeros((16,), jnp.float32)

        def zrow(i, carry):
            for t in range(d // 16):
                zbuf[i, pl.ds(t * 16, 16)] = zv
            return carry

        lax.fori_loop(0, rpt, zrow, 0)
        pltpu.sync_copy(zbuf, acc.at[pl.ds(sid * rpt, rpt)])
        pltpu.sync_copy(dst_h.at[pl.ds(cbase, nch)], idxd)
        plsc.subcore_barrier()

        def lstart(j, b):
            pltpu.async_copy(h_h.at[pl.ds(ebase + j * _C, _C)],
                             hbuf.at[b], seml)

        def lwait(j, b):
            pltpu.make_async_copy(h_h.at[pl.ds(ebase + j * _C, _C)],
                                  hbuf.at[b], seml).wait()

        def astart(j, b):
            pltpu.async_copy(hbuf.at[b], acc.at[idxd.at[j]], sema, add=True)

        def await_(j, b):
            pltpu.make_async_copy(h_h.at[pl.ds(ebase + j * _C, _C)],
                                  hbuf.at[b], sema).wait()

        ngr = (nch - 1) // 4
        for b in range(4):
            lstart(b, b)

        def group(g, carry):
            j0 = 4 * g
            for b in range(4):
                lwait(j0 + b, b)
                astart(j0 + b, b)
            for b in range(4):
                await_(j0 + b, b)
                nxt = j0 + 4 + b
                pred = nxt < nch

                @pl.when(pred)
                def _():
                    lstart(nxt, b)
            return carry

        lax.fori_loop(0, ngr, group, 0)
        lwait(nch - 1, 0)
        pltpu.sync_copy(hbuf.at[0], acc.at[idxd.at[nch - 1]], add=True)

        plsc.subcore_barrier()
        pltpu.sync_copy(acc.at[pl.ds(sid * rpt, rpt)],
                        out_h.at[cid, pl.ds(sid * rpt, rpt)])

    return run(h, dstg)


def kernel(x, edge_index,
           enc_W0, enc_b0, enc_W1, enc_b1, enc_W2, enc_b2, enc_W3, enc_b3,
           dec_W0, dec_b0, dec_W1, dec_b1, dec_W2, dec_b2, dec_W3, dec_b3):
    nn, din = x.shape
    ne = edge_index.shape[1]
    lat = enc_W3.shape[1]

    src = edge_index[0]
    dst = edge_index[1]
    dstg = dst.reshape(ne // _C, _C)
    srcg = src.reshape(ne // _C, _C)

    f32 = jnp.float32
    wa = enc_W0[:din] - enc_W0[din:]
    wb = enc_W0[din:]
    b0 = enc_b0.reshape(1, -1)
    b1 = enc_b1.reshape(1, -1)
    b2 = enc_b2.reshape(1, -1)
    w3p = jnp.concatenate([enc_W3, jnp.zeros((enc_W3.shape[0], 16 - lat), f32)],
                          axis=1)
    b3p = jnp.concatenate(
        [enc_b3, jnp.ones((1,), f32), jnp.zeros((16 - lat - 1,), f32)]
    ).reshape(1, 16)

    eap = jnp.pad(dec_W0[:lat] - dec_W0[lat:], ((0, 8 - lat), (0, 0)))
    ebp = jnp.pad(dec_W0[lat:], ((0, 8 - lat), (0, 0)))
    db0 = dec_b0.reshape(1, -1)
    db1 = dec_b1.reshape(1, -1)
    db2 = dec_b2.reshape(1, -1)
    db3 = dec_b3.reshape(1, -1)

    w1b, w2b, w3pb = _blkdiag(enc_W1), _blkdiag(enc_W2), _blkdiag(w3p)
    b1b, b2b, b3pb = (jnp.tile(b1, (1, _PK)), jnp.tile(b2, (1, _PK)),
                      jnp.tile(b3p, (1, _PK)))
    dw1b, dw2b = _blkdiag(dec_W1), _blkdiag(dec_W2)
    db1b, db2b = jnp.tile(db1, (1, _PK)), jnp.tile(db2, (1, _PK))
    eab, ebb = _blkdiag(eap), _blkdiag(ebp)
    db0b = jnp.tile(db0, (1, _PK))

    u, v = _node_tables_tc(x, wa, wb, b0)
    gu, gv = _sc_gather_pair(u, v, dstg, srcg)
    gu8 = gu.reshape(ne // _PK, _PK * 32)
    gv8 = gv.reshape(ne // _PK, _PK * 32)
    e4 = _edge_mlp_x3_tc(
        gu8, gv8,
        [_split_bf16(w1b), _split_bf16(w2b), _split_bf16(w3pb)],
        [b1b, b2b, b3pb])
    p = _sc_scatter_add(e4.reshape(ne, 16), dstg, nn)
    z8, cnt = _dec_tables_tc(p, lat)
    zd, zs = _sc_gather_pair(z8, z8, dstg, srcg)
    zd8 = zd.reshape(ne // _PK, _PK * 8)
    zs8 = zs.reshape(ne // _PK, _PK * 8)
    h3 = _edge_mlp_tc(zd8, zs8, dw1b, db1b, dw2b, db2b,
                      prec=lax.Precision.DEFAULT,
                      first=(eab, ebb, db0b))
    pd = _sc_scatter_add(h3.reshape(ne, 32), dstg, nn)
    return _final_tc(pd, cnt, dec_W3, db3)
